# baseline (speedup 1.0000x reference)
import functools

import jax
import jax.numpy as jnp
from jax import lax
from jax.experimental import pallas as pl
from jax.experimental.pallas import tpu as pltpu
from jax.experimental.pallas import tpu_sc as plsc

N_ATOMS = 262144
N_FRAMES = 4096
N_ELEMS = 94
E_PAD = 128
N_BINS = N_FRAMES * E_PAD
NC = 2
NS = 16
CHUNK = N_ATOMS // (NC * NS)
ROWS = CHUNK // E_PAD
HIST_SLICE = N_BINS // NS
NS_ITERS = 18
NSCAT = 8
CHF = 512

_mesh = plsc.VectorSubcoreMesh(core_axis_name="c", subcore_axis_name="s")


def _sc_hist_body(z_hbm, f_hbm, out_hbm, zbuf, fbuf, fill, hist,
                  zsem, fsem, ssem):
    cid = lax.axis_index("c")
    sid = lax.axis_index("s")
    base = cid * (N_ATOMS // NC) + sid * CHUNK

    zcp = pltpu.async_copy(z_hbm.at[pl.ds(base, CHUNK)], zbuf, zsem)
    fcp = pltpu.async_copy(f_hbm.at[pl.ds(base, CHUNK)], fbuf, fsem)

    zero16 = jnp.zeros((16,), jnp.float32)

    def _zfill(j, _):
        for k in range(8):
            fill[pl.ds(j * 128 + k * 16, 16)] = zero16
        return 0

    lax.fori_loop(0, CHUNK // 128, _zfill, 0)
    hist_flat = hist
    for k in range(HIST_SLICE // CHUNK):
        pltpu.sync_copy(fill, hist_flat.at[pl.ds(sid * HIST_SLICE + k * CHUNK, CHUNK)])

    zcp.wait()
    fcp.wait()

    ones16 = jnp.full((16,), 1.0, jnp.float32)

    def _ifill(j, _):
        for k in range(8):
            off = j * 128 + k * 16
            z = zbuf[pl.ds(off, 16)]
            f = fbuf[pl.ds(off, 16)]
            zbuf[pl.ds(off, 16)] = f * E_PAD + z - 1
            fill[pl.ds(off, 16)] = ones16
        return 0

    lax.fori_loop(0, CHUNK // 128, _ifill, 0)

    plsc.subcore_barrier()

    SCAT = CHUNK // NSCAT
    for k in range(NSCAT):
        pltpu.async_copy(
            fill.at[pl.ds(k * SCAT, SCAT)],
            hist_flat.at[zbuf.at[pl.ds(k * SCAT, SCAT)]],
            ssem, add=True)
    for k in range(NSCAT):
        pltpu.make_async_copy(
            fill.at[pl.ds(k * SCAT, SCAT)],
            hist_flat.at[zbuf.at[pl.ds(k * SCAT, SCAT)]],
            ssem).wait()

    plsc.subcore_barrier()

    pltpu.sync_copy(
        hist.at[pl.ds(sid * HIST_SLICE, HIST_SLICE)],
        out_hbm.at[pl.ds(cid * N_BINS + sid * HIST_SLICE, HIST_SLICE)],
    )


_sc_hist = functools.partial(
    pl.kernel,
    out_type=jax.ShapeDtypeStruct((NC * N_BINS,), jnp.float32),
    mesh=_mesh,
    scratch_types=[
        pltpu.VMEM((CHUNK,), jnp.int32),
        pltpu.VMEM((CHUNK,), jnp.int32),
        pltpu.VMEM((CHUNK,), jnp.float32),
        pltpu.VMEM_SHARED((N_BINS,), jnp.float32),
        pltpu.SemaphoreType.DMA,
        pltpu.SemaphoreType.DMA,
        pltpu.SemaphoreType.DMA,
    ],
)(_sc_hist_body)


def _mm(a, b):
    return lax.dot_general(a, b, (((1,), (0,)), ((), ())),
                           preferred_element_type=jnp.float32)


def _fit_body(parts_ref, y_ref, out_ref, xbuf, pbuf, sems):
    def _copies(i):
        par = i % 2
        return (
            pltpu.make_async_copy(
                parts_ref.at[pl.ds(i * CHF, CHF)], pbuf.at[par, 0],
                sems.at[par, 0]),
            pltpu.make_async_copy(
                parts_ref.at[pl.ds(N_FRAMES + i * CHF, CHF)], pbuf.at[par, 1],
                sems.at[par, 1]),
        )

    pend = _copies(0)
    pend[0].start()
    pend[1].start()
    G = None
    b = None
    for i in range(N_FRAMES // CHF):
        par = i % 2
        nxt = None
        if (i + 1) * CHF < N_FRAMES:
            nxt = _copies(i + 1)
            nxt[0].start()
            nxt[1].start()
        pend[0].wait()
        pend[1].wait()
        pend = nxt
        xc = pbuf[par, 0] + pbuf[par, 1]
        xbuf[i * CHF:(i + 1) * CHF] = xc
        yc = y_ref[i * CHF:(i + 1) * CHF]
        Gc = lax.dot_general(xc, xc, (((0,), (0,)), ((), ())),
                             preferred_element_type=jnp.float32)
        bc = lax.dot_general(xc, yc, (((0,), (0,)), ((), ())),
                             preferred_element_type=jnp.float32)
        G = Gc if G is None else G + Gc
        b = bc if b is None else b + bc
    x = xbuf[...]
    y = y_ref[...]
    s = jnp.max(jnp.sum(jnp.abs(G), axis=1))
    a = 1.0 / (s * s)
    r2 = lax.broadcasted_iota(jnp.int32, (E_PAD, 2 * E_PAD), 0)
    c2 = lax.broadcasted_iota(jnp.int32, (E_PAD, 2 * E_PAD), 1)
    eye_l = jnp.where(r2 == c2, 1.0, 0.0).astype(jnp.float32)
    m_v = jnp.where(c2 == E_PAD, 1.0, 0.0).astype(jnp.float32)
    sgn = m_v - jnp.where(c2 < E_PAD, 1.0, 0.0).astype(jnp.float32)
    gb = jnp.concatenate(
        [G, jnp.broadcast_to(b, (E_PAD, 1)),
         jnp.zeros((E_PAD, E_PAD - 1), jnp.float32)], axis=1) * a
    C = eye_l + sgn * _mm(G, gb)

    for _ in range(NS_ITERS):
        C = _mm(C[:, :E_PAD], C) + C * m_v
    beta = C[:, E_PAD:E_PAD + 1]
    res = y - _mm(x, beta)
    out_ref[...] = res.reshape(N_FRAMES // E_PAD, E_PAD)


_tc_fit = pl.pallas_call(
    _fit_body,
    out_shape=jax.ShapeDtypeStruct((N_FRAMES // E_PAD, E_PAD), jnp.float32),
    in_specs=[
        pl.BlockSpec(memory_space=pl.ANY),
        pl.BlockSpec(memory_space=pltpu.VMEM),
    ],
    out_specs=pl.BlockSpec(memory_space=pltpu.VMEM),
    scratch_shapes=[
        pltpu.VMEM((N_FRAMES, E_PAD), jnp.float32),
        pltpu.VMEM((2, 2, CHF, E_PAD), jnp.float32),
        pltpu.SemaphoreType.DMA((2, 2)),
    ],
)


def kernel(energy, Z, frame_ids):
    parts = _sc_hist(Z, frame_ids)
    parts2d = parts.reshape(NC * N_FRAMES, E_PAD)
    out = _tc_fit(parts2d, energy.reshape(N_FRAMES, 1))
    return out.reshape(N_FRAMES)


# speedup vs baseline: 1.0875x; 1.0875x over previous
import functools

import jax
import jax.numpy as jnp
from jax import lax
from jax.experimental import pallas as pl
from jax.experimental.pallas import tpu as pltpu
from jax.experimental.pallas import tpu_sc as plsc

N_ATOMS = 262144
N_FRAMES = 4096
N_ELEMS = 94
E_PAD = 128
N_BINS = N_FRAMES * E_PAD
NC = 2
NS = 16
CHUNK = N_ATOMS // (NC * NS)
ROWS = CHUNK // E_PAD
HIST_SLICE = N_BINS // NS
NS_ITERS = 18
NSCAT = 8
CHF = 512

_mesh = plsc.VectorSubcoreMesh(core_axis_name="c", subcore_axis_name="s")


def _sc_hist_body(z_hbm, f_hbm, out_hbm, zbuf, fbuf, fill, hist,
                  zsem, fsem, ssem):
    cid = lax.axis_index("c")
    sid = lax.axis_index("s")
    base = cid * (N_ATOMS // NC) + sid * CHUNK

    zcp = pltpu.async_copy(z_hbm.at[pl.ds(base, CHUNK)], zbuf, zsem)
    fcp = pltpu.async_copy(f_hbm.at[pl.ds(base, CHUNK)], fbuf, fsem)

    zero16 = jnp.zeros((16,), jnp.float32)

    def _zfill(j, _):
        for k in range(8):
            fill[pl.ds(j * 128 + k * 16, 16)] = zero16
        return 0

    lax.fori_loop(0, CHUNK // 128, _zfill, 0)
    hist_flat = hist
    for k in range(HIST_SLICE // CHUNK):
        pltpu.sync_copy(fill, hist_flat.at[pl.ds(sid * HIST_SLICE + k * CHUNK, CHUNK)])

    zcp.wait()
    fcp.wait()

    ones16 = jnp.full((16,), 1.0, jnp.float32)

    def _ifill(j, _):
        for k in range(8):
            off = j * 128 + k * 16
            z = zbuf[pl.ds(off, 16)]
            f = fbuf[pl.ds(off, 16)]
            zbuf[pl.ds(off, 16)] = f * E_PAD + z - 1
            fill[pl.ds(off, 16)] = ones16
        return 0

    lax.fori_loop(0, CHUNK // 128, _ifill, 0)

    plsc.subcore_barrier()

    SCAT = CHUNK // NSCAT
    for k in range(NSCAT):
        pltpu.async_copy(
            fill.at[pl.ds(k * SCAT, SCAT)],
            hist_flat.at[zbuf.at[pl.ds(k * SCAT, SCAT)]],
            ssem, add=True)
    for k in range(NSCAT):
        pltpu.make_async_copy(
            fill.at[pl.ds(k * SCAT, SCAT)],
            hist_flat.at[zbuf.at[pl.ds(k * SCAT, SCAT)]],
            ssem).wait()

    plsc.subcore_barrier()

    pltpu.sync_copy(
        hist.at[pl.ds(sid * HIST_SLICE, HIST_SLICE)],
        out_hbm.at[pl.ds(cid * N_BINS + sid * HIST_SLICE, HIST_SLICE)],
    )


_sc_hist = functools.partial(
    pl.kernel,
    out_type=jax.ShapeDtypeStruct((NC * N_BINS,), jnp.float32),
    mesh=_mesh,
    scratch_types=[
        pltpu.VMEM((CHUNK,), jnp.int32),
        pltpu.VMEM((CHUNK,), jnp.int32),
        pltpu.VMEM((CHUNK,), jnp.float32),
        pltpu.VMEM_SHARED((N_BINS,), jnp.float32),
        pltpu.SemaphoreType.DMA,
        pltpu.SemaphoreType.DMA,
        pltpu.SemaphoreType.DMA,
    ],
)(_sc_hist_body)


def _mm(a, b):
    return lax.dot_general(a, b, (((1,), (0,)), ((), ())),
                           preferred_element_type=jnp.float32)


def _fit_body(parts_ref, y_ref, out_ref):
    x = parts_ref[:N_FRAMES] + parts_ref[N_FRAMES:]
    y = y_ref[...]
    G = lax.dot_general(x, x, (((0,), (0,)), ((), ())),
                        preferred_element_type=jnp.float32)
    b = lax.dot_general(x, y, (((0,), (0,)), ((), ())),
                        preferred_element_type=jnp.float32)
    s = jnp.max(jnp.sum(jnp.abs(G), axis=1))
    a = 1.0 / (s * s)
    r2 = lax.broadcasted_iota(jnp.int32, (E_PAD, 2 * E_PAD), 0)
    c2 = lax.broadcasted_iota(jnp.int32, (E_PAD, 2 * E_PAD), 1)
    eye_l = jnp.where(r2 == c2, 1.0, 0.0).astype(jnp.float32)
    m_v = jnp.where(c2 == E_PAD, 1.0, 0.0).astype(jnp.float32)
    sgn = m_v - jnp.where(c2 < E_PAD, 1.0, 0.0).astype(jnp.float32)
    gb = jnp.concatenate(
        [G, jnp.broadcast_to(b, (E_PAD, 1)),
         jnp.zeros((E_PAD, E_PAD - 1), jnp.float32)], axis=1) * a
    C = eye_l + sgn * _mm(G, gb)

    for _ in range(NS_ITERS):
        C = _mm(C[:, :E_PAD], C) + C * m_v
    beta = C[:, E_PAD:E_PAD + 1]
    res = y - _mm(x, beta)
    out_ref[...] = res.reshape(N_FRAMES // E_PAD, E_PAD)


_tc_fit = pl.pallas_call(
    _fit_body,
    out_shape=jax.ShapeDtypeStruct((N_FRAMES // E_PAD, E_PAD), jnp.float32),
    in_specs=[
        pl.BlockSpec(memory_space=pltpu.VMEM),
        pl.BlockSpec(memory_space=pltpu.VMEM),
    ],
    out_specs=pl.BlockSpec(memory_space=pltpu.VMEM),
)


def kernel(energy, Z, frame_ids):
    parts = _sc_hist(Z, frame_ids)
    parts2d = parts.reshape(NC * N_FRAMES, E_PAD)
    out = _tc_fit(parts2d, energy.reshape(N_FRAMES, 1))
    return out.reshape(N_FRAMES)


# speedup vs baseline: 1.0890x; 1.0013x over previous
import functools

import jax
import jax.numpy as jnp
from jax import lax
from jax.experimental import pallas as pl
from jax.experimental.pallas import tpu as pltpu
from jax.experimental.pallas import tpu_sc as plsc

N_ATOMS = 262144
N_FRAMES = 4096
N_ELEMS = 94
E_PAD = 128
N_BINS = N_FRAMES * E_PAD
NC = 2
NS = 16
CHUNK = N_ATOMS // (NC * NS)
HIST_SLICE = N_BINS // NS
NS_ITERS = 18
NSCAT = 8

_mesh = plsc.VectorSubcoreMesh(core_axis_name="c", subcore_axis_name="s")


def _sc_hist_body(z_hbm, f_hbm, out_hbm, zbuf, fbuf, fill, hist,
                  zsem, fsem, ssem):
    cid = lax.axis_index("c")
    sid = lax.axis_index("s")
    base = cid * (N_ATOMS // NC) + sid * CHUNK

    zcp = pltpu.async_copy(z_hbm.at[pl.ds(base, CHUNK)], zbuf, zsem)
    fcp = pltpu.async_copy(f_hbm.at[pl.ds(base, CHUNK)], fbuf, fsem)

    zero16 = jnp.zeros((16,), jnp.float32)

    def _zfill(j, _):
        for k in range(8):
            fill[pl.ds(j * 128 + k * 16, 16)] = zero16
        return 0

    lax.fori_loop(0, CHUNK // 128, _zfill, 0)
    hist_flat = hist
    for k in range(HIST_SLICE // CHUNK):
        pltpu.sync_copy(fill, hist_flat.at[pl.ds(sid * HIST_SLICE + k * CHUNK, CHUNK)])

    zcp.wait()
    fcp.wait()

    ones16 = jnp.full((16,), 1.0, jnp.float32)

    def _ifill(j, _):
        for k in range(8):
            off = j * 128 + k * 16
            z = zbuf[pl.ds(off, 16)]
            f = fbuf[pl.ds(off, 16)]
            zbuf[pl.ds(off, 16)] = f * E_PAD + z - 1
            fill[pl.ds(off, 16)] = ones16
        return 0

    lax.fori_loop(0, CHUNK // 128, _ifill, 0)

    plsc.subcore_barrier()

    SCAT = CHUNK // NSCAT
    for k in range(NSCAT):
        pltpu.async_copy(
            fill.at[pl.ds(k * SCAT, SCAT)],
            hist_flat.at[zbuf.at[pl.ds(k * SCAT, SCAT)]],
            ssem, add=True)
    for k in range(NSCAT):
        pltpu.make_async_copy(
            fill.at[pl.ds(k * SCAT, SCAT)],
            hist_flat.at[zbuf.at[pl.ds(k * SCAT, SCAT)]],
            ssem).wait()

    plsc.subcore_barrier()

    pltpu.sync_copy(
        hist.at[pl.ds(sid * HIST_SLICE, HIST_SLICE)],
        out_hbm.at[pl.ds(cid * N_BINS + sid * HIST_SLICE, HIST_SLICE)],
    )


_sc_hist = functools.partial(
    pl.kernel,
    out_type=jax.ShapeDtypeStruct((NC * N_BINS,), jnp.float32),
    mesh=_mesh,
    scratch_types=[
        pltpu.VMEM((CHUNK,), jnp.int32),
        pltpu.VMEM((CHUNK,), jnp.int32),
        pltpu.VMEM((CHUNK,), jnp.float32),
        pltpu.VMEM_SHARED((N_BINS,), jnp.float32),
        pltpu.SemaphoreType.DMA,
        pltpu.SemaphoreType.DMA,
        pltpu.SemaphoreType.DMA,
    ],
)(_sc_hist_body)


def _mm(a, b):
    return lax.dot_general(a, b, (((1,), (0,)), ((), ())),
                           preferred_element_type=jnp.float32)


def _fit_body(parts_ref, y_ref, out_ref):
    x = parts_ref[:N_FRAMES] + parts_ref[N_FRAMES:]
    y = y_ref[...]
    G = lax.dot_general(x, x, (((0,), (0,)), ((), ())),
                        preferred_element_type=jnp.float32)
    b = lax.dot_general(x, y, (((0,), (0,)), ((), ())),
                        preferred_element_type=jnp.float32)
    s = jnp.max(jnp.sum(jnp.abs(G), axis=1))
    a = 1.0 / (s * s)
    r2 = lax.broadcasted_iota(jnp.int32, (E_PAD, 2 * E_PAD), 0)
    c2 = lax.broadcasted_iota(jnp.int32, (E_PAD, 2 * E_PAD), 1)
    eye_l = jnp.where(r2 == c2, 1.0, 0.0).astype(jnp.float32)
    m_v = jnp.where(c2 == E_PAD, 1.0, 0.0).astype(jnp.float32)
    sgn = m_v - jnp.where(c2 < E_PAD, 1.0, 0.0).astype(jnp.float32)
    gb = jnp.concatenate(
        [G, jnp.broadcast_to(b, (E_PAD, 1)),
         jnp.zeros((E_PAD, E_PAD - 1), jnp.float32)], axis=1) * a
    C = eye_l + sgn * _mm(G, gb)

    for _ in range(NS_ITERS):
        C = _mm(C[:, :E_PAD], C) + C * m_v
    beta = C[:, E_PAD:E_PAD + 1]
    res = y - _mm(x, beta)
    out_ref[...] = res.reshape(N_FRAMES // E_PAD, E_PAD)


_tc_fit = pl.pallas_call(
    _fit_body,
    out_shape=jax.ShapeDtypeStruct((N_FRAMES // E_PAD, E_PAD), jnp.float32),
    in_specs=[
        pl.BlockSpec(memory_space=pltpu.VMEM),
        pl.BlockSpec(memory_space=pltpu.VMEM),
    ],
    out_specs=pl.BlockSpec(memory_space=pltpu.VMEM),
)


def kernel(energy, Z, frame_ids):
    parts = _sc_hist(Z, frame_ids)
    parts2d = parts.reshape(NC * N_FRAMES, E_PAD)
    out = _tc_fit(parts2d, energy.reshape(N_FRAMES, 1))
    return out.reshape(N_FRAMES)
